# baseline (speedup 1.0000x reference)
import jax
import jax.numpy as jnp
from jax import lax
from jax.experimental import pallas as pl
from jax.experimental.pallas import tpu as pltpu
from jax.experimental.pallas import tpu_sc as plsc

N_NODES = 10000
D_FEAT = 128
N_EDGES = 320000
NUM_CORES = 2
NUM_SUBCORES = 16
NW = NUM_CORES * NUM_SUBCORES
COLS = D_FEAT // NW
K = 4096
NCHUNKS = 80
E_PAD = K * NCHUNKS
LANES = 16


def _sc_colslice(in_hbm, meta_hbm, out_hbm,
                 in0, in1, in2, in3, ac0, ac1, ac2, ac3,
                 meta_a, meta_b, msem_a, msem_b):
    cid = lax.axis_index("c")
    sid = lax.axis_index("s")
    wid = cid * NUM_SUBCORES + sid

    ins = [in0, in1, in2, in3]
    accs = [ac0, ac1, ac2, ac3]

    pltpu.async_copy(meta_hbm.at[0], meta_a, msem_a)
    for c in range(COLS):
        pltpu.sync_copy(in_hbm.at[pl.ds((wid * COLS + c) * N_NODES, N_NODES)],
                        ins[c])

    zeros16 = jnp.zeros((LANES,), jnp.float32)

    @pl.loop(0, N_NODES // LANES)
    def _zero(i):
        o = pl.ds(i * LANES, LANES)
        for c in range(COLS):
            accs[c][o] = zeros16

    def _process(meta_v):
        @pl.loop(0, K // LANES, unroll=16)
        def _group(g):
            o = pl.ds(g * LANES, LANES)
            s16 = meta_v[0, o]
            t16 = meta_v[1, o]
            w16 = (plsc.bitcast(meta_v[2, o], jnp.float32) *
                   plsc.bitcast(meta_v[3, o], jnp.float32))
            for c in range(COLS):
                vals = plsc.load_gather(ins[c], [s16])
                plsc.addupdate_scatter(accs[c], [t16], vals * w16)

    @pl.loop(0, NCHUNKS // 2)
    def _pair(p):
        ch0 = 2 * p
        pltpu.make_async_copy(meta_hbm.at[ch0], meta_a, msem_a).wait()
        pltpu.async_copy(meta_hbm.at[ch0 + 1], meta_b, msem_b)
        _process(meta_a)
        pltpu.make_async_copy(meta_hbm.at[ch0 + 1], meta_b, msem_b).wait()

        @pl.when(p + 1 < NCHUNKS // 2)
        def _prefetch_next():
            pltpu.async_copy(meta_hbm.at[ch0 + 2], meta_a, msem_a)

        _process(meta_b)

    for c in range(COLS):
        pltpu.sync_copy(accs[c],
                        out_hbm.at[pl.ds((wid * COLS + c) * N_NODES, N_NODES)])


@jax.jit
def _graph_conv(inT, meta):
    mesh = plsc.VectorSubcoreMesh(core_axis_name="c", subcore_axis_name="s")
    outT = pl.kernel(
        _sc_colslice,
        out_type=jax.ShapeDtypeStruct((NW * COLS * N_NODES,), jnp.float32),
        mesh=mesh,
        compiler_params=pltpu.CompilerParams(needs_layout_passes=False),
        scratch_types=(
            [pltpu.VMEM((N_NODES,), jnp.float32) for _ in range(2 * COLS)] +
            [pltpu.VMEM((4, K), jnp.int32) for _ in range(2)] +
            [pltpu.SemaphoreType.DMA, pltpu.SemaphoreType.DMA]
        ),
    )(inT, meta)
    return outT


def _pad1(x):
    return jnp.concatenate([x, jnp.zeros((E_PAD - N_EDGES,), x.dtype)])


def kernel(input, eidx, enorm, esgn):
    eidx = eidx.astype(jnp.int32)
    meta = jnp.stack([
        _pad1(eidx[0]),
        _pad1(eidx[1]),
        lax.bitcast_convert_type(_pad1(enorm), jnp.int32),
        lax.bitcast_convert_type(_pad1(esgn), jnp.int32),
    ])
    meta = meta.reshape(4, NCHUNKS, K).transpose(1, 0, 2)
    inT = input.T.reshape(-1)
    outT = _graph_conv(inT, meta)
    return outT.reshape(D_FEAT, N_NODES).T


# speedup vs baseline: 1.1800x; 1.1800x over previous
import jax
import jax.numpy as jnp
from jax import lax
from jax.experimental import pallas as pl
from jax.experimental.pallas import tpu as pltpu
from jax.experimental.pallas import tpu_sc as plsc

N_NODES = 10000
D_FEAT = 128
N_EDGES = 320000
NUM_CORES = 2
NUM_SUBCORES = 16
NW = NUM_CORES * NUM_SUBCORES
CHUNK = 128
CHUNKS_PER_TILE = 80
GROUP = 8
E_PAD = NW * CHUNKS_PER_TILE * CHUNK
IO_TILES = 10
ROWS_PER_TILE = N_NODES // IO_TILES
ZROWS = 40
LANES = 16


def _sc_scatter(input_hbm, sidx_hbm, tidx_hbm, en_hbm, es_hbm, part_hbm,
                accum_sh, sidx_v, tidx_v, en_v, es_v, rows0_v, rows1_v,
                stage_v, gsem0, gsem1, ssem0, ssem1):
    cid = lax.axis_index("c")
    sid = lax.axis_index("s")
    wid = cid * NUM_SUBCORES + sid

    zeros16 = jnp.zeros((LANES,), jnp.float32)

    @pl.loop(0, ZROWS)
    def _zero_rows(i):
        for j in range(D_FEAT // LANES):
            stage_v[i, pl.ds(j * LANES, LANES)] = zeros16

    row0 = sid * ROWS_PER_TILE

    @pl.when(sid < IO_TILES)
    def _zero_accum():
        @pl.loop(0, ROWS_PER_TILE // ZROWS)
        def _z(k):
            pltpu.sync_copy(stage_v, accum_sh.at[pl.ds(row0 + k * ZROWS, ZROWS)])

    plsc.subcore_barrier()

    rows = [rows0_v, rows1_v]
    gsems = [gsem0, gsem1]
    ssems = [ssem0, ssem1]

    def _scale_rows(rows_b, k):
        @pl.loop(0, CHUNK // LANES)
        def _scale(q):
            qsl = pl.ds(q * LANES, LANES)
            wv = en_v[k, qsl] * es_v[k, qsl]
            for ii in range(LANES):
                i = q * LANES + ii
                w = wv[ii]
                for j in range(D_FEAT // LANES):
                    sl = pl.ds(j * LANES, LANES)
                    rows_b[i, sl] = rows_b[i, sl] * w

    @pl.loop(0, CHUNKS_PER_TILE // GROUP)
    def _group(g):
        gsl = pl.ds(g * GROUP, GROUP)
        pltpu.sync_copy(sidx_hbm.at[wid].at[gsl], sidx_v)
        pltpu.sync_copy(tidx_hbm.at[wid].at[gsl], tidx_v)
        pltpu.sync_copy(en_hbm.at[wid].at[gsl], en_v)
        pltpu.sync_copy(es_hbm.at[wid].at[gsl], es_v)

        gat = [None, None]
        scat = [None, None]
        gat[0] = pltpu.async_copy(input_hbm.at[sidx_v.at[0]], rows[0], gsems[0])
        for k in range(GROUP):
            b = k & 1
            nb = 1 - b
            if k + 1 < GROUP:
                if scat[nb] is not None:
                    scat[nb].wait()
                gat[nb] = pltpu.async_copy(
                    input_hbm.at[sidx_v.at[k + 1]], rows[nb], gsems[nb])
            gat[b].wait()
            _scale_rows(rows[b], k)
            scat[b] = pltpu.async_copy(
                rows[b], accum_sh.at[tidx_v.at[k]], ssems[b], add=True)
        scat[0].wait()
        scat[1].wait()

    plsc.subcore_barrier()

    @pl.when(sid < IO_TILES)
    def _writeback():
        @pl.loop(0, ROWS_PER_TILE // ZROWS)
        def _w(k):
            sl = pl.ds(row0 + k * ZROWS, ZROWS)
            pltpu.sync_copy(accum_sh.at[sl], stage_v)
            pltpu.sync_copy(stage_v, part_hbm.at[cid].at[sl])


@jax.jit
def _graph_conv(input, sidx, tidx, en, es):
    mesh = plsc.VectorSubcoreMesh(core_axis_name="c", subcore_axis_name="s")
    partials = pl.kernel(
        _sc_scatter,
        out_type=jax.ShapeDtypeStruct((NUM_CORES, N_NODES, D_FEAT), jnp.float32),
        mesh=mesh,
        scratch_types=[
            pltpu.VMEM_SHARED((N_NODES, D_FEAT), jnp.float32),
            pltpu.VMEM((GROUP, CHUNK), jnp.int32),
            pltpu.VMEM((GROUP, CHUNK), jnp.int32),
            pltpu.VMEM((GROUP, CHUNK), jnp.float32),
            pltpu.VMEM((GROUP, CHUNK), jnp.float32),
            pltpu.VMEM((CHUNK, D_FEAT), jnp.float32),
            pltpu.VMEM((CHUNK, D_FEAT), jnp.float32),
            pltpu.VMEM((ZROWS, D_FEAT), jnp.float32),
            pltpu.SemaphoreType.DMA,
            pltpu.SemaphoreType.DMA,
            pltpu.SemaphoreType.DMA,
            pltpu.SemaphoreType.DMA,
        ],
    )(input, sidx, tidx, en, es)

    def _combine(p_ref, o_ref):
        o_ref[...] = p_ref[0] + p_ref[1]

    return pl.pallas_call(
        _combine,
        grid=(10,),
        in_specs=[pl.BlockSpec((NUM_CORES, N_NODES // 10, D_FEAT),
                               lambda i: (0, i, 0))],
        out_specs=pl.BlockSpec((N_NODES // 10, D_FEAT), lambda i: (i, 0)),
        out_shape=jax.ShapeDtypeStruct((N_NODES, D_FEAT), jnp.float32),
    )(partials)


def _pad3(x, fill):
    pad = E_PAD - N_EDGES
    x = jnp.concatenate([x, jnp.full((pad,), fill, x.dtype)])
    return x.reshape(NW, CHUNKS_PER_TILE, CHUNK)


def kernel(input, eidx, enorm, esgn):
    eidx = eidx.astype(jnp.int32)
    sidx = _pad3(eidx[0], 0)
    tidx = _pad3(eidx[1], 0)
    en = _pad3(enorm, 0.0)
    es = _pad3(esgn, 0.0)
    return _graph_conv(input, sidx, tidx, en, es)
